# baseline (speedup 1.0000x reference)
import functools

import jax
import jax.numpy as jnp
from jax import lax
from jax.experimental import pallas as pl
from jax.experimental.pallas import tpu as pltpu
from jax.experimental.pallas import tpu_sc as plsc

_NT = 16
_NC = 2
_C = 128
_GROUP = 768
_GC = _GROUP // _C
_BS = 32
_NSLAB = 8
_NBUF = 3
_NIB = 5
_HB = 8


def _sc_call(xt, rcw_all, w_all, bias_b, O, I, n_groups):
    mesh = plsc.VectorSubcoreMesh(core_axis_name="c", subcore_axis_name="s")
    o_slice = O // _NT

    @functools.partial(
        pl.kernel,
        out_type=jax.ShapeDtypeStruct((_NSLAB * O, _BS), jnp.float32),
        mesh=mesh,
        compiler_params=pltpu.CompilerParams(use_tc_tiling_on_sc=False),
        scratch_types=[
            pltpu.VMEM((_NIB, 2 * _HB, _C), jnp.int32),
            pltpu.VMEM((_NIB, _GROUP), jnp.float32),
            pltpu.VMEM((_NBUF, _GROUP, _BS), jnp.float32),
            pltpu.VMEM_SHARED((O, _BS), jnp.float32),
            pltpu.SemaphoreType.DMA((_NIB,)),
            pltpu.SemaphoreType.DMA((_NBUF,)),
            pltpu.SemaphoreType.DMA((_NBUF,)),
        ],
    )
    def sck(xt_h, rcw_h, w_h, bias_h, out_h, rcw, wbuf, gbuf, acc, isem, gsem, ssem):
        c = lax.axis_index("c")
        s = lax.axis_index("s")

        def load_idx(g, r):
            blk = pl.multiple_of((s * n_groups + g) * 2 * _HB, 8)
            pltpu.async_copy(rcw_h.at[pl.ds(blk, 2 * _HB)], rcw.at[r], isem.at[r])
            ebase = pl.multiple_of((s * n_groups + g) * _GROUP, 8)
            pltpu.async_copy(w_h.at[pl.ds(ebase, _GROUP)], wbuf.at[r], isem.at[r])

        def fire_gathers(slab, q, r):
            pltpu.make_async_copy(
                rcw_h.at[pl.ds(0, 2 * _HB)], rcw.at[r], isem.at[r]
            ).wait()
            pltpu.make_async_copy(
                w_h.at[pl.ds(0, _GROUP)], wbuf.at[r], isem.at[r]
            ).wait()
            off = jnp.full((16,), slab * I, jnp.int32)
            for i in range(_GC):
                for jj in range(_C // 16):
                    rcw[r, i, pl.ds(jj * 16, 16)] = rcw[r, i, pl.ds(jj * 16, 16)] + off
            for i in range(_GC):
                pltpu.async_copy(
                    xt_h.at[rcw.at[r, i]],
                    gbuf.at[q, pl.ds(i * _C, _C)],
                    gsem.at[q],
                )

        def wait_scatter(p):
            pltpu.make_async_copy(
                out_h.at[pl.ds(0, _GROUP)], gbuf.at[p], ssem.at[p]
            ).wait()

        def proc(p, r):
            pltpu.make_async_copy(
                out_h.at[pl.ds(0, _GROUP)], gbuf.at[p], gsem.at[p]
            ).wait()

            @plsc.parallel_loop(0, _GROUP // 16, unroll=2)
            def _(k16):
                kb = k16 * 16
                wvec = wbuf[r, pl.ds(kb, 16)]
                for l in range(16):
                    wv = jnp.full((16,), wvec[l], jnp.float32)
                    for j in range(_BS // 16):
                        gbuf[p, kb + l, pl.ds(j * 16, 16)] = (
                            gbuf[p, kb + l, pl.ds(j * 16, 16)] * wv
                        )

            for i in range(_GC):
                pltpu.async_copy(
                    gbuf.at[p, pl.ds(i * _C, _C)],
                    acc.at[rcw.at[r, _HB + i]],
                    ssem.at[p],
                    add=True,
                )

        for sl in range(_NSLAB // _NC):
            slab = c * (_NSLAB // _NC) + sl
            obase = pl.multiple_of(s * o_slice, 8)
            pltpu.sync_copy(
                bias_h.at[pl.ds(obase, o_slice)], acc.at[pl.ds(obase, o_slice)]
            )
            plsc.subcore_barrier()

            load_idx(0, 0)
            load_idx(1, 1)
            load_idx(2, 2)
            fire_gathers(slab, 0, 0)
            fire_gathers(slab, 1, 1)
            proc(0, 0)
            fire_gathers(slab, 2, 2)
            load_idx(3, 3)

            def slot(g, carry):
                p = lax.rem(g, _NBUF)
                r = lax.rem(g, _NIB)
                q2 = lax.rem(g + 2, _NBUF)
                r2 = lax.rem(g + 2, _NIB)
                r3 = lax.rem(g + 3, _NIB)
                proc(p, r)
                wait_scatter(q2)
                fire_gathers(slab, q2, r2)
                load_idx(g + 3, r3)
                return carry

            lax.fori_loop(1, n_groups - 3, slot, 0)
            g = n_groups - 3
            proc(g % _NBUF, g % _NIB)
            wait_scatter((g + 2) % _NBUF)
            fire_gathers(slab, (g + 2) % _NBUF, (g + 2) % _NIB)
            g = n_groups - 2
            proc(g % _NBUF, g % _NIB)
            wait_scatter((g + 2) % _NBUF)
            g = n_groups - 1
            proc(g % _NBUF, g % _NIB)
            wait_scatter((g - 1) % _NBUF)
            wait_scatter(g % _NBUF)

            plsc.subcore_barrier()
            pltpu.sync_copy(
                acc.at[pl.ds(obase, o_slice)],
                out_h.at[pl.ds(pl.multiple_of(slab * O + s * o_slice, 8), o_slice)],
            )

    return sck(xt, rcw_all, w_all, bias_b)


def kernel(x, row_idxs, col_idxs, weights, bias):
    B, I = x.shape
    O = bias.shape[0]
    NNZ = row_idxs.shape[0]
    per_tile = -(-NNZ // (_NT * _GROUP)) * _GROUP
    n_groups = per_tile // _GROUP
    if n_groups < 6:
        n_groups = 6
        per_tile = n_groups * _GROUP
    NP = per_tile * _NT
    pad = NP - NNZ
    ngt = NP // _GROUP
    rows_p = jnp.pad(row_idxs, (0, pad)).reshape(ngt, _GC, _C)
    cols_p = jnp.pad(col_idxs, (0, pad)).reshape(ngt, _GC, _C)
    rows_p = jnp.pad(rows_p, ((0, 0), (0, _HB - _GC), (0, 0)))
    cols_p = jnp.pad(cols_p, ((0, 0), (0, _HB - _GC), (0, 0)))
    w_all = jnp.pad(weights, (0, pad))
    rcw_all = jnp.concatenate([rows_p, cols_p], axis=1).reshape(ngt * 2 * _HB, _C)
    xt = x.reshape(_NSLAB, _BS, I).transpose(0, 2, 1).reshape(_NSLAB * I, _BS)
    bias_b = jnp.broadcast_to(bias, (O, _BS))
    out = _sc_call(xt, rcw_all, w_all, bias_b, O, I, n_groups)
    return out.reshape(_NSLAB, O, _BS).transpose(0, 2, 1).reshape(B, O)


# speedup vs baseline: 1.0138x; 1.0138x over previous
import functools

import jax
import jax.numpy as jnp
from jax import lax
from jax.experimental import pallas as pl
from jax.experimental.pallas import tpu as pltpu
from jax.experimental.pallas import tpu_sc as plsc

_NT = 16
_NC = 2
_C = 128
_GROUP = 512
_GC = _GROUP // _C
_BS = 32
_NSLAB = 8
_NBUF = 3
_NIB = 5
_HB = ((_GC + 3) // 4) * 4


def _sc_call(xt, rcw_all, w_all, bias_b, O, I, n_groups):
    mesh = plsc.VectorSubcoreMesh(core_axis_name="c", subcore_axis_name="s")
    o_slice = O // _NT

    @functools.partial(
        pl.kernel,
        out_type=jax.ShapeDtypeStruct((_NSLAB * O, _BS), jnp.float32),
        mesh=mesh,
        compiler_params=pltpu.CompilerParams(use_tc_tiling_on_sc=False),
        scratch_types=[
            pltpu.VMEM((_NIB, 2 * _HB, _C), jnp.int32),
            pltpu.VMEM((_NIB, _GROUP), jnp.float32),
            pltpu.VMEM((_NBUF, _GROUP, _BS), jnp.float32),
            pltpu.VMEM_SHARED((O, _BS), jnp.float32),
            pltpu.SemaphoreType.DMA((_NIB,)),
            pltpu.SemaphoreType.DMA((_NBUF,)),
            pltpu.SemaphoreType.DMA((_NBUF,)),
        ],
    )
    def sck(xt_h, rcw_h, w_h, bias_h, out_h, rcw, wbuf, gbuf, acc, isem, gsem, ssem):
        c = lax.axis_index("c")
        s = lax.axis_index("s")

        def load_idx(g, r):
            blk = pl.multiple_of((s * n_groups + g) * 2 * _HB, 8)
            pltpu.async_copy(rcw_h.at[pl.ds(blk, 2 * _HB)], rcw.at[r], isem.at[r])
            ebase = pl.multiple_of((s * n_groups + g) * _GROUP, 8)
            pltpu.async_copy(w_h.at[pl.ds(ebase, _GROUP)], wbuf.at[r], isem.at[r])

        def fire_gathers(slab, q, r):
            pltpu.make_async_copy(
                rcw_h.at[pl.ds(0, 2 * _HB)], rcw.at[r], isem.at[r]
            ).wait()
            pltpu.make_async_copy(
                w_h.at[pl.ds(0, _GROUP)], wbuf.at[r], isem.at[r]
            ).wait()
            off = jnp.full((16,), slab * I, jnp.int32)
            for i in range(_GC):
                for jj in range(_C // 16):
                    rcw[r, i, pl.ds(jj * 16, 16)] = rcw[r, i, pl.ds(jj * 16, 16)] + off
            for i in range(_GC):
                pltpu.async_copy(
                    xt_h.at[rcw.at[r, i]],
                    gbuf.at[q, pl.ds(i * _C, _C)],
                    gsem.at[q],
                )

        def wait_scatter(p):
            pltpu.make_async_copy(
                out_h.at[pl.ds(0, _GROUP)], gbuf.at[p], ssem.at[p]
            ).wait()

        def proc(p, r):
            pltpu.make_async_copy(
                out_h.at[pl.ds(0, _GROUP)], gbuf.at[p], gsem.at[p]
            ).wait()

            @plsc.parallel_loop(0, _GROUP // 16, unroll=2)
            def _(k16):
                kb = k16 * 16
                wvec = wbuf[r, pl.ds(kb, 16)]
                for l in range(16):
                    wv = jnp.full((16,), wvec[l], jnp.float32)
                    for j in range(_BS // 16):
                        gbuf[p, kb + l, pl.ds(j * 16, 16)] = (
                            gbuf[p, kb + l, pl.ds(j * 16, 16)] * wv
                        )

            for i in range(_GC):
                pltpu.async_copy(
                    gbuf.at[p, pl.ds(i * _C, _C)],
                    acc.at[rcw.at[r, _HB + i]],
                    ssem.at[p],
                    add=True,
                )

        for sl in range(_NSLAB // _NC):
            slab = c * (_NSLAB // _NC) + sl
            obase = pl.multiple_of(s * o_slice, 8)
            pltpu.sync_copy(
                bias_h.at[pl.ds(obase, o_slice)], acc.at[pl.ds(obase, o_slice)]
            )
            plsc.subcore_barrier()

            load_idx(0, 0)
            load_idx(1, 1)
            load_idx(2, 2)
            fire_gathers(slab, 0, 0)
            fire_gathers(slab, 1, 1)
            proc(0, 0)
            fire_gathers(slab, 2, 2)
            load_idx(3, 3)

            def slot(g, carry):
                p = lax.rem(g, _NBUF)
                r = lax.rem(g, _NIB)
                q2 = lax.rem(g + 2, _NBUF)
                r2 = lax.rem(g + 2, _NIB)
                r3 = lax.rem(g + 3, _NIB)
                proc(p, r)
                wait_scatter(q2)
                fire_gathers(slab, q2, r2)
                load_idx(g + 3, r3)
                return carry

            lax.fori_loop(1, n_groups - 3, slot, 0)
            g = n_groups - 3
            proc(g % _NBUF, g % _NIB)
            wait_scatter((g + 2) % _NBUF)
            fire_gathers(slab, (g + 2) % _NBUF, (g + 2) % _NIB)
            g = n_groups - 2
            proc(g % _NBUF, g % _NIB)
            wait_scatter((g + 2) % _NBUF)
            g = n_groups - 1
            proc(g % _NBUF, g % _NIB)
            wait_scatter((g - 1) % _NBUF)
            wait_scatter(g % _NBUF)

            plsc.subcore_barrier()
            pltpu.sync_copy(
                acc.at[pl.ds(obase, o_slice)],
                out_h.at[pl.ds(pl.multiple_of(slab * O + s * o_slice, 8), o_slice)],
            )

    return sck(xt, rcw_all, w_all, bias_b)


def kernel(x, row_idxs, col_idxs, weights, bias):
    B, I = x.shape
    O = bias.shape[0]
    NNZ = row_idxs.shape[0]
    per_tile = -(-NNZ // (_NT * _GROUP)) * _GROUP
    n_groups = per_tile // _GROUP
    if n_groups < 6:
        n_groups = 6
        per_tile = n_groups * _GROUP
    NP = per_tile * _NT
    pad = NP - NNZ
    ngt = NP // _GROUP
    rows_p = jnp.pad(row_idxs, (0, pad)).reshape(ngt, _GC, _C)
    cols_p = jnp.pad(col_idxs, (0, pad)).reshape(ngt, _GC, _C)
    rows_p = jnp.pad(rows_p, ((0, 0), (0, _HB - _GC), (0, 0)))
    cols_p = jnp.pad(cols_p, ((0, 0), (0, _HB - _GC), (0, 0)))
    w_all = jnp.pad(weights, (0, pad))
    rcw_all = jnp.concatenate([rows_p, cols_p], axis=1).reshape(ngt * 2 * _HB, _C)
    xt = x.reshape(_NSLAB, _BS, I).transpose(0, 2, 1).reshape(_NSLAB * I, _BS)
    bias_b = jnp.broadcast_to(bias, (O, _BS))
    out = _sc_call(xt, rcw_all, w_all, bias_b, O, I, n_groups)
    return out.reshape(_NSLAB, O, _BS).transpose(0, 2, 1).reshape(B, O)


# speedup vs baseline: 1.0204x; 1.0066x over previous
import functools

import jax
import jax.numpy as jnp
from jax import lax
from jax.experimental import pallas as pl
from jax.experimental.pallas import tpu as pltpu
from jax.experimental.pallas import tpu_sc as plsc

_NT = 16
_NC = 2
_C = 128
_GROUP = 512
_GC = _GROUP // _C
_BS = 32
_NSLAB = 8
_NBUF = 4
_NIB = 5
_HB = ((_GC + 3) // 4) * 4


def _sc_call(xt, rcw_all, w_all, bias_b, O, I, n_groups):
    mesh = plsc.VectorSubcoreMesh(core_axis_name="c", subcore_axis_name="s")
    o_slice = O // _NT

    @functools.partial(
        pl.kernel,
        out_type=jax.ShapeDtypeStruct((_NSLAB * O, _BS), jnp.float32),
        mesh=mesh,
        compiler_params=pltpu.CompilerParams(use_tc_tiling_on_sc=False),
        scratch_types=[
            pltpu.VMEM((_NIB, 2 * _HB, _C), jnp.int32),
            pltpu.VMEM((_NIB, _GROUP), jnp.float32),
            pltpu.VMEM((_NBUF, _GROUP, _BS), jnp.float32),
            pltpu.VMEM_SHARED((O, _BS), jnp.float32),
            pltpu.SemaphoreType.DMA((_NIB,)),
            pltpu.SemaphoreType.DMA((_NBUF,)),
            pltpu.SemaphoreType.DMA((_NBUF,)),
        ],
    )
    def sck(xt_h, rcw_h, w_h, bias_h, out_h, rcw, wbuf, gbuf, acc, isem, gsem, ssem):
        c = lax.axis_index("c")
        s = lax.axis_index("s")

        def load_idx(g, r):
            blk = pl.multiple_of((s * n_groups + g) * 2 * _HB, 8)
            pltpu.async_copy(rcw_h.at[pl.ds(blk, 2 * _HB)], rcw.at[r], isem.at[r])
            ebase = pl.multiple_of((s * n_groups + g) * _GROUP, 8)
            pltpu.async_copy(w_h.at[pl.ds(ebase, _GROUP)], wbuf.at[r], isem.at[r])

        def fire_gathers(slab, q, r):
            pltpu.make_async_copy(
                rcw_h.at[pl.ds(0, 2 * _HB)], rcw.at[r], isem.at[r]
            ).wait()
            pltpu.make_async_copy(
                w_h.at[pl.ds(0, _GROUP)], wbuf.at[r], isem.at[r]
            ).wait()
            off = jnp.full((16,), slab * I, jnp.int32)
            for i in range(_GC):
                for jj in range(_C // 16):
                    rcw[r, i, pl.ds(jj * 16, 16)] = rcw[r, i, pl.ds(jj * 16, 16)] + off
            for i in range(_GC):
                pltpu.async_copy(
                    xt_h.at[rcw.at[r, i]],
                    gbuf.at[q, pl.ds(i * _C, _C)],
                    gsem.at[q],
                )

        def wait_scatter(p):
            pltpu.make_async_copy(
                out_h.at[pl.ds(0, _GROUP)], gbuf.at[p], ssem.at[p]
            ).wait()

        def proc(p, r):
            pltpu.make_async_copy(
                out_h.at[pl.ds(0, _GROUP)], gbuf.at[p], gsem.at[p]
            ).wait()

            @plsc.parallel_loop(0, _GROUP // 16, unroll=2)
            def _(k16):
                kb = k16 * 16
                wvec = wbuf[r, pl.ds(kb, 16)]
                for l in range(16):
                    wv = jnp.full((16,), wvec[l], jnp.float32)
                    for j in range(_BS // 16):
                        gbuf[p, kb + l, pl.ds(j * 16, 16)] = (
                            gbuf[p, kb + l, pl.ds(j * 16, 16)] * wv
                        )

            for i in range(_GC):
                pltpu.async_copy(
                    gbuf.at[p, pl.ds(i * _C, _C)],
                    acc.at[rcw.at[r, _HB + i]],
                    ssem.at[p],
                    add=True,
                )

        for sl in range(_NSLAB // _NC):
            slab = c * (_NSLAB // _NC) + sl
            obase = pl.multiple_of(s * o_slice, 8)
            pltpu.sync_copy(
                bias_h.at[pl.ds(obase, o_slice)], acc.at[pl.ds(obase, o_slice)]
            )
            plsc.subcore_barrier()

            load_idx(0, 0)
            load_idx(1, 1)
            load_idx(2, 2)
            fire_gathers(slab, 0, 0)
            fire_gathers(slab, 1, 1)
            proc(0, 0)
            fire_gathers(slab, 2, 2)
            load_idx(3, 3)

            def slot(g, carry):
                p = lax.rem(g, _NBUF)
                r = lax.rem(g, _NIB)
                q2 = lax.rem(g + 2, _NBUF)
                r2 = lax.rem(g + 2, _NIB)
                r3 = lax.rem(g + 3, _NIB)
                proc(p, r)
                wait_scatter(lax.rem(g + _NBUF - 1, _NBUF))
                fire_gathers(slab, q2, r2)
                load_idx(g + 3, r3)
                return carry

            lax.fori_loop(1, n_groups - 3, slot, 0)
            g = n_groups - 3
            proc(g % _NBUF, g % _NIB)
            wait_scatter((g - 1) % _NBUF)
            fire_gathers(slab, (g + 2) % _NBUF, (g + 2) % _NIB)
            g = n_groups - 2
            proc(g % _NBUF, g % _NIB)
            wait_scatter((g - 1) % _NBUF)
            g = n_groups - 1
            proc(g % _NBUF, g % _NIB)
            wait_scatter((g - 1) % _NBUF)
            wait_scatter(g % _NBUF)

            plsc.subcore_barrier()
            pltpu.sync_copy(
                acc.at[pl.ds(obase, o_slice)],
                out_h.at[pl.ds(pl.multiple_of(slab * O + s * o_slice, 8), o_slice)],
            )

    return sck(xt, rcw_all, w_all, bias_b)


def kernel(x, row_idxs, col_idxs, weights, bias):
    B, I = x.shape
    O = bias.shape[0]
    NNZ = row_idxs.shape[0]
    per_tile = -(-NNZ // (_NT * _GROUP)) * _GROUP
    n_groups = per_tile // _GROUP
    if n_groups < 6:
        n_groups = 6
        per_tile = n_groups * _GROUP
    NP = per_tile * _NT
    pad = NP - NNZ
    ngt = NP // _GROUP
    rows_p = jnp.pad(row_idxs, (0, pad)).reshape(ngt, _GC, _C)
    cols_p = jnp.pad(col_idxs, (0, pad)).reshape(ngt, _GC, _C)
    rows_p = jnp.pad(rows_p, ((0, 0), (0, _HB - _GC), (0, 0)))
    cols_p = jnp.pad(cols_p, ((0, 0), (0, _HB - _GC), (0, 0)))
    w_all = jnp.pad(weights, (0, pad))
    rcw_all = jnp.concatenate([rows_p, cols_p], axis=1).reshape(ngt * 2 * _HB, _C)
    xt = x.reshape(_NSLAB, _BS, I).transpose(0, 2, 1).reshape(_NSLAB * I, _BS)
    bias_b = jnp.broadcast_to(bias, (O, _BS))
    out = _sc_call(xt, rcw_all, w_all, bias_b, O, I, n_groups)
    return out.reshape(_NSLAB, O, _BS).transpose(0, 2, 1).reshape(B, O)
